# baseline (speedup 1.0000x reference)
import functools
import jax
import jax.numpy as jnp
from jax import lax
from jax.experimental import pallas as pl
from jax.experimental.pallas import tpu as pltpu
from jax.experimental.pallas import tpu_sc as plsc

NC = 2
NS = 16
NW = NC * NS
B = 4096
F = 26
D = 16
PB = B // NW
PF = PB * F
CH = 128
NCH = PF // CH
GRP = PB // 16

_mesh = plsc.VectorSubcoreMesh(core_axis_name="c", subcore_axis_name="s")


@functools.partial(
    pl.kernel,
    out_type=jax.ShapeDtypeStruct((B,), jnp.float32),
    mesh=_mesh,
    scratch_types=[
        pltpu.VMEM((NCH, CH), jnp.int32),
        pltpu.VMEM((PF, D), jnp.float32),
        pltpu.VMEM((PF,), jnp.float32),
        pltpu.VMEM((PB,), jnp.float32),
        pltpu.SemaphoreType.DMA,
        pltpu.SemaphoreType.DMA,
    ],
    compiler_params=pltpu.CompilerParams(
        use_tc_tiling_on_sc=False, needs_layout_passes=False),
)
def _fm_kernel(feat_hbm, emb_hbm, coeff_hbm, out_hbm,
               idx_v, rows_v, cof_v, out_v, sem_e, sem_c):
    wid = lax.axis_index("s") * NC + lax.axis_index("c")

    pltpu.sync_copy(feat_hbm.at[wid], idx_v)

    @pl.loop(0, NCH)
    def _fire(j):
        pltpu.async_copy(emb_hbm.at[idx_v.at[j]],
                         rows_v.at[pl.ds(j * CH, CH)], sem_e)
        pltpu.async_copy(coeff_hbm.at[idx_v.at[j]],
                         cof_v.at[pl.ds(j * CH, CH)], sem_c)

    pltpu.make_async_copy(emb_hbm.at[pl.ds(0, PF)], rows_v, sem_e).wait()
    pltpu.make_async_copy(coeff_hbm.at[pl.ds(0, PF)], cof_v, sem_c).wait()

    @pl.loop(0, GRP)
    def _grp(g):
        lane = lax.iota(jnp.int32, 16)
        mask_b = (lane >= 6).astype(jnp.float32)
        out_acc = jnp.zeros((16,), jnp.float32)
        for s in range(16):
            base = (g * 16 + s) * F
            v = rows_v[base]
            acc = v
            acc2 = v * v
            for f in range(1, F):
                v = rows_v[base + f]
                acc = acc + v
                acc2 = acc2 + v * v
            c_a = cof_v[pl.ds(base, 16)]
            c_b = cof_v[pl.ds(base + 10, 16)]
            t = 0.5 * (acc * acc - acc2) + c_a + c_b * mask_b
            tot = jnp.sum(t)
            out_acc = jnp.where(lane == s, tot, out_acc)
        out_v[pl.ds(g * 16, 16)] = out_acc

    pltpu.sync_copy(out_v, out_hbm.at[pl.ds(wid * PB, PB)])


@jax.jit
def kernel(features, feature_embedding, feature_coeff, bias):
    feat = features.astype(jnp.int32).reshape(NW, NCH, CH)
    coeff = feature_coeff.reshape(-1)
    out = _fm_kernel(feat, feature_embedding, coeff)
    return out + bias


# speedup vs baseline: 1.0001x; 1.0001x over previous
import functools
import jax
import jax.numpy as jnp
from jax import lax
from jax.experimental import pallas as pl
from jax.experimental.pallas import tpu as pltpu
from jax.experimental.pallas import tpu_sc as plsc

NC = 2
NS = 16
NW = NC * NS
B = 4096
F = 26
D = 16
PB = B // NW
PF = PB * F
CH = 128
NCH = PF // CH
GRP = PB // 16

_mesh = plsc.VectorSubcoreMesh(core_axis_name="c", subcore_axis_name="s")


@functools.partial(
    pl.kernel,
    out_type=jax.ShapeDtypeStruct((B,), jnp.float32),
    mesh=_mesh,
    scratch_types=[
        pltpu.VMEM((PB, F), jnp.int32),
        pltpu.VMEM((PF,), jnp.int32),
        pltpu.VMEM((PF, D), jnp.float32),
        pltpu.VMEM((PF,), jnp.float32),
        pltpu.VMEM((PB,), jnp.float32),
        pltpu.SemaphoreType.DMA,
        pltpu.SemaphoreType.DMA,
    ],
    compiler_params=pltpu.CompilerParams(
        use_tc_tiling_on_sc=False, needs_layout_passes=False),
)
def _fm_kernel(feat_hbm, emb_hbm, coeff_hbm, out_hbm,
               idx_v, fidx_v, rows_v, cof_v, out_v, sem_e, sem_c):
    wid = lax.axis_index("s") * NC + lax.axis_index("c")

    pltpu.sync_copy(feat_hbm.at[pl.ds(wid * PB, PB), :], idx_v)

    @pl.loop(0, PF // 16)
    def _flat(k):
        p = k * 16 + lax.iota(jnp.int32, 16)
        row = p // F
        col = p - row * F
        fidx_v[pl.ds(k * 16, 16)] = plsc.load_gather(idx_v, [row, col])

    @pl.loop(0, NCH)
    def _fire(j):
        pltpu.async_copy(emb_hbm.at[fidx_v.at[pl.ds(j * CH, CH)]],
                         rows_v.at[pl.ds(j * CH, CH)], sem_e)
        pltpu.async_copy(coeff_hbm.at[fidx_v.at[pl.ds(j * CH, CH)]],
                         cof_v.at[pl.ds(j * CH, CH)], sem_c)

    pltpu.make_async_copy(emb_hbm.at[pl.ds(0, PF)], rows_v, sem_e).wait()
    pltpu.make_async_copy(coeff_hbm.at[pl.ds(0, PF)], cof_v, sem_c).wait()

    @pl.loop(0, GRP)
    def _grp(g):
        lane = lax.iota(jnp.int32, 16)
        mask_b = (lane >= 6).astype(jnp.float32)
        out_acc = jnp.zeros((16,), jnp.float32)
        for s in range(16):
            base = (g * 16 + s) * F
            v = rows_v[base]
            acc = v
            acc2 = v * v
            for f in range(1, F):
                v = rows_v[base + f]
                acc = acc + v
                acc2 = acc2 + v * v
            c_a = cof_v[pl.ds(base, 16)]
            c_b = cof_v[pl.ds(base + 10, 16)]
            t = 0.5 * (acc * acc - acc2) + c_a + c_b * mask_b
            tot = jnp.sum(t)
            out_acc = jnp.where(lane == s, tot, out_acc)
        out_v[pl.ds(g * 16, 16)] = out_acc

    pltpu.sync_copy(out_v, out_hbm.at[pl.ds(wid * PB, PB)])


@jax.jit
def kernel(features, feature_embedding, feature_coeff, bias):
    out = _fm_kernel(features.astype(jnp.int32), feature_embedding,
                     feature_coeff.reshape(-1))
    return out + bias
